# baseline (speedup 1.0000x reference)
import jax
import jax.numpy as jnp
from jax import lax
from jax.experimental import pallas as pl
from jax.experimental.pallas import tpu as pltpu
from jax.experimental.pallas import tpu_sc as plsc

NC = 2
NS = 16
NW = NC * NS

B = 16384
D = 128
CH = 64
BPW = B // NW
NCHUNK = BPW // CH
NBUF = 14
NTOT = 2 * NCHUNK


def _body(idx1_hbm, idx2_hbm, w1_hbm, w2_hbm, out1_hbm, out2_hbm,
          idx1_v, idx2_v, *bufs_and_sems):
    bufs = bufs_and_sems[:NBUF]
    gsems = bufs_and_sems[NBUF:2 * NBUF]
    ssems = bufs_and_sems[2 * NBUF:]
    wid = lax.axis_index("s") * NC + lax.axis_index("c")
    base = wid * BPW

    i1d = pltpu.async_copy(idx1_hbm.at[pl.ds(base, BPW)], idx1_v, ssems[0])
    i2d = pltpu.async_copy(idx2_hbm.at[pl.ds(base, BPW)], idx2_v, ssems[1])

    chunks = [(idx1_v, w1_hbm, out1_hbm, j) for j in range(NCHUNK)] + \
             [(idx2_v, w2_hbm, out2_hbm, j) for j in range(NCHUNK)]

    def fire(c, b):
        idx_v, w_hbm, _, j = chunks[c]
        return pltpu.async_copy(
            w_hbm.at[idx_v.at[pl.ds(j * CH, CH)]], bufs[b], gsems[b])

    def store(c, b):
        _, _, out_hbm, j = chunks[c]
        return pltpu.async_copy(
            bufs[b], out_hbm.at[pl.ds(base + j * CH, CH)], ssems[b])

    i1d.wait()
    gds = [fire(c, c) for c in range(NCHUNK)]
    i2d.wait()
    gds += [fire(c, c) for c in range(NCHUNK, NBUF)]
    sds = []
    for c in range(NBUF):
        gds[c].wait()
        sds.append(store(c, c))
    for c in range(NBUF, NTOT):
        b = c - NBUF
        sds[b].wait()
        fire(c, b).wait()
        sds[b] = store(c, b)
    for d in sds:
        d.wait()


def kernel(word1, word2, W1, W2):
    idx1 = word1.astype(jnp.int32)
    idx2 = word2.astype(jnp.int32)

    mesh = plsc.VectorSubcoreMesh(core_axis_name="c", subcore_axis_name="s")
    out1, out2 = pl.kernel(
        _body,
        out_type=(
            jax.ShapeDtypeStruct((B, D), jnp.float32),
            jax.ShapeDtypeStruct((B, D), jnp.float32),
        ),
        mesh=mesh,
        scratch_types=(
            [pltpu.VMEM((BPW,), jnp.int32)] * 2
            + [pltpu.VMEM((CH, D), jnp.float32) for _ in range(NBUF)]
            + [pltpu.SemaphoreType.DMA for _ in range(2 * NBUF)]
        ),
    )(idx1, idx2, W1, W2)
    return (out1, out2)


# speedup vs baseline: 1.0304x; 1.0304x over previous
import jax
import jax.numpy as jnp
from jax import lax
from jax.experimental import pallas as pl
from jax.experimental.pallas import tpu as pltpu
from jax.experimental.pallas import tpu_sc as plsc

NC = 2
NS = 16
NW = NC * NS

B = 16384
D = 128
BPW = B // NW

SIZES1 = (128, 128, 128, 128)
SIZES2 = (128, 128, 128, 96, 32)
NBUF = len(SIZES1) + len(SIZES2) - 1
NTOT = len(SIZES1) + len(SIZES2)


def _chunk_offsets(sizes):
    offs, o = [], 0
    for s in sizes:
        offs.append(o)
        o += s
    return offs


OFFS1 = _chunk_offsets(SIZES1)
OFFS2 = _chunk_offsets(SIZES2)


def _body(idx1_hbm, idx2_hbm, w1_hbm, w2_hbm, out1_hbm, out2_hbm,
          idx1_v, idx2_v, *bufs_and_sems):
    bufs = bufs_and_sems[:NBUF]
    gsems = bufs_and_sems[NBUF:2 * NBUF]
    ssems = bufs_and_sems[2 * NBUF:]
    wid = lax.axis_index("s") * NC + lax.axis_index("c")
    base = wid * BPW

    i1d = pltpu.async_copy(idx1_hbm.at[pl.ds(base, BPW)], idx1_v, ssems[0])
    i2d = pltpu.async_copy(idx2_hbm.at[pl.ds(base, BPW)], idx2_v, ssems[1])

    chunks = [(idx1_v, w1_hbm, out1_hbm, o, s) for o, s in zip(OFFS1, SIZES1)] + \
             [(idx2_v, w2_hbm, out2_hbm, o, s) for o, s in zip(OFFS2, SIZES2)]
    n1 = len(SIZES1)

    def fire(c, b):
        idx_v, w_hbm, _, o, s = chunks[c]
        return pltpu.async_copy(
            w_hbm.at[idx_v.at[pl.ds(o, s)]], bufs[b].at[pl.ds(0, s)],
            gsems[b])

    def store(c, b):
        _, _, out_hbm, o, s = chunks[c]
        return pltpu.async_copy(
            bufs[b].at[pl.ds(0, s)], out_hbm.at[pl.ds(base + o, s)],
            ssems[b])

    i1d.wait()
    gds = [fire(c, c) for c in range(n1)]
    i2d.wait()
    gds += [fire(c, c) for c in range(n1, NBUF)]
    sds = []
    for c in range(NBUF):
        gds[c].wait()
        sds.append(store(c, c))
    for c in range(NBUF, NTOT):
        b = c - NBUF
        sds[b].wait()
        fire(c, b).wait()
        sds[b] = store(c, b)
    for d in sds:
        d.wait()


def kernel(word1, word2, W1, W2):
    idx1 = word1.astype(jnp.int32)
    idx2 = word2.astype(jnp.int32)

    mesh = plsc.VectorSubcoreMesh(core_axis_name="c", subcore_axis_name="s")
    buf_sizes = list(SIZES1) + list(SIZES2[:-1])
    out1, out2 = pl.kernel(
        _body,
        out_type=(
            jax.ShapeDtypeStruct((B, D), jnp.float32),
            jax.ShapeDtypeStruct((B, D), jnp.float32),
        ),
        mesh=mesh,
        scratch_types=(
            [pltpu.VMEM((BPW,), jnp.int32)] * 2
            + [pltpu.VMEM((s, D), jnp.float32) for s in buf_sizes]
            + [pltpu.SemaphoreType.DMA for _ in range(2 * NBUF)]
        ),
    )(idx1, idx2, W1, W2)
    return (out1, out2)
